# Initial kernel scaffold; baseline (speedup 1.0000x reference)
import functools

import jax
import jax.numpy as jnp
from jax import lax
from jax.experimental import pallas as pl
from jax.experimental.pallas import tpu as pltpu
from jax.experimental.pallas import tpu_sc as plsc

N_NODES = 10000
N_EDGES = 320000
D_IN = 128
D_HID = 256

NC = 2
NS = 16
NW = NC * NS

CHUNK = 80
ROWS_PER_TILE = N_NODES // NS
ZROWS = 125

_mesh = plsc.VectorSubcoreMesh(core_axis_name="c", subcore_axis_name="s")


def _zero_vmem(zbuf):
  z16 = jnp.zeros((16,), jnp.float32)

  def zb(i, carry):
    r = i // 8
    col = (i % 8) * 16
    zbuf[r, pl.ds(col, 16)] = z16
    return carry

  lax.fori_loop(0, ZROWS * 8, zb, 0)


def _zero_acc_slice(zbuf, acc, s):
  def zc(k, carry):
    pltpu.sync_copy(zbuf, acc.at[pl.ds(s * ROWS_PER_TILE + k * ZROWS, ZROWS), :])
    return carry

  lax.fori_loop(0, ROWS_PER_TILE // ZROWS, zc, 0)


def _agg_loop(src_hbm, dst_hbm, tbl_hbm, acc, src_v, dst_v, rows_v, sem,
              base0, n_chunks, deg_v=None):
  ones16 = jnp.ones((16,), jnp.float32)

  def step(i, carry):
    base = base0 + i * CHUNK
    pltpu.sync_copy(src_hbm.at[pl.ds(base, CHUNK)], src_v)
    pltpu.sync_copy(dst_hbm.at[pl.ds(base, CHUNK)], dst_v)
    pltpu.async_copy(tbl_hbm.at[src_v], rows_v, sem).wait()
    pltpu.sync_copy(rows_v, acc.at[dst_v], add=True)
    if deg_v is not None:
      for j in range(CHUNK // 16):
        dvec = dst_v[pl.ds(j * 16, 16)]
        plsc.addupdate_scatter(deg_v, [dvec], ones16)
    return carry

  lax.fori_loop(0, n_chunks, step, 0)


def _sc_agg1_body(src_hbm, dst_hbm, x_hbm, sums_hbm, degs_hbm,
                  src_v, dst_v, rows_v, deg_v, zbuf, acc, sem):
  c = lax.axis_index("c")
  s = lax.axis_index("s")

  _zero_vmem(zbuf)
  z16 = jnp.zeros((16,), jnp.float32)

  def zd(i, carry):
    deg_v[pl.ds(i * 16, 16)] = z16
    return carry

  lax.fori_loop(0, N_NODES // 16, zd, 0)
  _zero_acc_slice(zbuf, acc, s)
  plsc.subcore_barrier()

  edges_per_tile = N_EDGES // NW
  base0 = c * (N_EDGES // NC) + s * edges_per_tile
  _agg_loop(src_hbm, dst_hbm, x_hbm, acc, src_v, dst_v, rows_v, sem,
            base0, edges_per_tile // CHUNK, deg_v=deg_v)

  plsc.subcore_barrier()
  row0 = s * ROWS_PER_TILE
  pltpu.sync_copy(acc.at[pl.ds(row0, ROWS_PER_TILE), :],
                  sums_hbm.at[c, pl.ds(row0, ROWS_PER_TILE), :])
  pltpu.sync_copy(deg_v, degs_hbm.at[c * NS + s])


_sc_agg1 = functools.partial(
    pl.kernel,
    out_type=(
        jax.ShapeDtypeStruct((NC, N_NODES, D_IN), jnp.float32),
        jax.ShapeDtypeStruct((NW, N_NODES), jnp.float32),
    ),
    mesh=_mesh,
    scratch_types=[
        pltpu.VMEM((CHUNK,), jnp.int32),
        pltpu.VMEM((CHUNK,), jnp.int32),
        pltpu.VMEM((CHUNK, D_IN), jnp.float32),
        pltpu.VMEM((N_NODES,), jnp.float32),
        pltpu.VMEM((ZROWS, D_IN), jnp.float32),
        pltpu.VMEM_SHARED((N_NODES, D_IN), jnp.float32),
        pltpu.SemaphoreType.DMA,
    ],
)(_sc_agg1_body)


def _sc_agg2_body(src_hbm, dst_hbm, tbls_hbm, sums_hbm,
                  src_v, dst_v, rows_v, zbuf, acc, sem):
  c = lax.axis_index("c")
  s = lax.axis_index("s")

  _zero_vmem(zbuf)
  _zero_acc_slice(zbuf, acc, s)
  plsc.subcore_barrier()

  edges_per_tile = N_EDGES // NS
  base0 = s * edges_per_tile
  _agg_loop(src_hbm, dst_hbm, tbls_hbm.at[c], acc, src_v, dst_v, rows_v, sem,
            base0, edges_per_tile // CHUNK)

  plsc.subcore_barrier()
  row0 = s * ROWS_PER_TILE
  pltpu.sync_copy(acc.at[pl.ds(row0, ROWS_PER_TILE), :],
                  sums_hbm.at[c, pl.ds(row0, ROWS_PER_TILE), :])


_sc_agg2 = functools.partial(
    pl.kernel,
    out_type=jax.ShapeDtypeStruct((NC, N_NODES, D_IN), jnp.float32),
    mesh=_mesh,
    scratch_types=[
        pltpu.VMEM((CHUNK,), jnp.int32),
        pltpu.VMEM((CHUNK,), jnp.int32),
        pltpu.VMEM((CHUNK, D_IN), jnp.float32),
        pltpu.VMEM((ZROWS, D_IN), jnp.float32),
        pltpu.VMEM_SHARED((N_NODES, D_IN), jnp.float32),
        pltpu.SemaphoreType.DMA,
    ],
)(_sc_agg2_body)


BLK = 1000


def _tc1_body(x_ref, sa_ref, sb_ref, degp_ref, w1rel_ref, b1_ref, w1root_ref,
              lo_ref, hi_ref):
  deg = jnp.sum(degp_ref[...], axis=0)
  inv = 1.0 / jnp.maximum(deg, 1.0)
  mean = (sa_ref[...][0] + sb_ref[...][0]) * inv[:, None]
  h = jnp.dot(mean, w1rel_ref[...], preferred_element_type=jnp.float32)
  h = h + jnp.dot(x_ref[...], w1root_ref[...], preferred_element_type=jnp.float32)
  h = h + b1_ref[...]
  h = jnp.maximum(h, 0.0)
  lo_ref[...] = h[:, :D_IN]
  hi_ref[...] = h[:, D_IN:]


def _tc2_body(mlo_ref, mhi_ref, degp_ref, hlo_ref, hhi_ref,
              w2rel_ref, b2_ref, w2root_ref, out_ref):
  deg = jnp.sum(degp_ref[...], axis=0)
  inv = 1.0 / jnp.maximum(deg, 1.0)
  mean = jnp.concatenate([mlo_ref[...][0], mhi_ref[...][0]], axis=1) * inv[:, None]
  h = jnp.concatenate([hlo_ref[...], hhi_ref[...]], axis=1)
  out = jnp.dot(mean, w2rel_ref[...], preferred_element_type=jnp.float32)
  out = out + jnp.dot(h, w2root_ref[...], preferred_element_type=jnp.float32)
  out_ref[...] = out + b2_ref[...]


def _tc1(x, sums1, degp, W1_rel, b1, W1_root):
  grid = (N_NODES // BLK,)
  return pl.pallas_call(
      _tc1_body,
      grid=grid,
      in_specs=[
          pl.BlockSpec((BLK, D_IN), lambda i: (i, 0)),
          pl.BlockSpec((1, BLK, D_IN), lambda i: (0, i, 0)),
          pl.BlockSpec((1, BLK, D_IN), lambda i: (1, i, 0)),
          pl.BlockSpec((NW, BLK), lambda i: (0, i)),
          pl.BlockSpec((D_IN, D_HID), lambda i: (0, 0)),
          pl.BlockSpec((1, D_HID), lambda i: (0, 0)),
          pl.BlockSpec((D_IN, D_HID), lambda i: (0, 0)),
      ],
      out_specs=[
          pl.BlockSpec((BLK, D_IN), lambda i: (i, 0)),
          pl.BlockSpec((BLK, D_IN), lambda i: (i, 0)),
      ],
      out_shape=[
          jax.ShapeDtypeStruct((N_NODES, D_IN), jnp.float32),
          jax.ShapeDtypeStruct((N_NODES, D_IN), jnp.float32),
      ],
  )(x, sums1, sums1, degp, W1_rel, b1.reshape(1, D_HID), W1_root)


def _tc2(sums2, degp, h_lo, h_hi, W2_rel, b2, W2_root):
  grid = (N_NODES // BLK,)
  return pl.pallas_call(
      _tc2_body,
      grid=grid,
      in_specs=[
          pl.BlockSpec((1, BLK, D_IN), lambda i: (0, i, 0)),
          pl.BlockSpec((1, BLK, D_IN), lambda i: (1, i, 0)),
          pl.BlockSpec((NW, BLK), lambda i: (0, i)),
          pl.BlockSpec((BLK, D_IN), lambda i: (i, 0)),
          pl.BlockSpec((BLK, D_IN), lambda i: (i, 0)),
          pl.BlockSpec((D_HID, D_HID), lambda i: (0, 0)),
          pl.BlockSpec((1, D_HID), lambda i: (0, 0)),
          pl.BlockSpec((D_HID, D_HID), lambda i: (0, 0)),
      ],
      out_specs=pl.BlockSpec((BLK, D_HID), lambda i: (i, 0)),
      out_shape=jax.ShapeDtypeStruct((N_NODES, D_HID), jnp.float32),
  )(sums2, sums2, degp, h_lo, h_hi, W2_rel, b2.reshape(1, D_HID), W2_root)


def kernel(x, edge_index, W1_rel, b1, W1_root, W2_rel, b2, W2_root):
  src = edge_index[0].astype(jnp.int32)
  dst = edge_index[1].astype(jnp.int32)

  sums1, degp = _sc_agg1(src, dst, x)
  h_lo, h_hi = _tc1(x, sums1, degp, W1_rel, b1, W1_root)
  tbls = jnp.stack([h_lo, h_hi], axis=0)
  sums2 = _sc_agg2(src, dst, tbls)
  return _tc2(sums2, degp, h_lo, h_hi, W2_rel, b2, W2_root)


# speedup vs baseline: 4.5747x; 4.5747x over previous
import functools

import jax
import jax.numpy as jnp
from jax import lax
from jax.experimental import pallas as pl
from jax.experimental.pallas import tpu as pltpu
from jax.experimental.pallas import tpu_sc as plsc

N_NODES = 10000
N_EDGES = 320000
D_IN = 128
D_HID = 256

NC = 2
NS = 16
NW = NC * NS

NPAD = 10240
CHUNK = 80
ROWS_PER_TILE = NPAD // NS
ZROWS = 128

_mesh = plsc.VectorSubcoreMesh(core_axis_name="c", subcore_axis_name="s")


def _zero_acc_slice(z2d_hbm, acc, s):
  row0 = s * ROWS_PER_TILE
  pltpu.sync_copy(z2d_hbm.at[pl.ds(row0, ROWS_PER_TILE), :],
                  acc.at[pl.ds(row0, ROWS_PER_TILE), :])


def _agg_loop(src_hbm, dst_hbm, tbl_hbm, acc, src_v, dst_v, rows_v, sem,
              base0, n_chunks, deg_v=None):
  ones16 = jnp.ones((16,), jnp.float32)

  def step(i, carry):
    base = base0 + i * CHUNK
    pltpu.sync_copy(src_hbm.at[pl.ds(base, CHUNK)], src_v)
    pltpu.sync_copy(dst_hbm.at[pl.ds(base, CHUNK)], dst_v)
    pltpu.async_copy(tbl_hbm.at[src_v], rows_v, sem).wait()
    pltpu.sync_copy(rows_v, acc.at[dst_v], add=True)
    if deg_v is not None:
      for j in range(CHUNK // 16):
        dvec = dst_v[pl.ds(j * 16, 16)]
        plsc.addupdate_scatter(deg_v, [dvec], ones16)
    return carry

  lax.fori_loop(0, n_chunks, step, 0)


def _sc_agg1_body(src_hbm, dst_hbm, x_hbm, z2d_hbm, z1d_hbm, sums_hbm, degs_hbm,
                  src_v, dst_v, rows_v, deg_v, acc, sem):
  c = lax.axis_index("c")
  s = lax.axis_index("s")

  pltpu.sync_copy(z1d_hbm, deg_v)
  _zero_acc_slice(z2d_hbm, acc, s)
  plsc.subcore_barrier()

  edges_per_tile = N_EDGES // NW
  base0 = c * (N_EDGES // NC) + s * edges_per_tile
  _agg_loop(src_hbm, dst_hbm, x_hbm, acc, src_v, dst_v, rows_v, sem,
            base0, edges_per_tile // CHUNK, deg_v=deg_v)

  plsc.subcore_barrier()
  row0 = s * ROWS_PER_TILE
  pltpu.sync_copy(acc.at[pl.ds(row0, ROWS_PER_TILE), :],
                  sums_hbm.at[c, pl.ds(row0, ROWS_PER_TILE), :])
  pltpu.sync_copy(deg_v, degs_hbm.at[c * NS + s])


_sc_agg1 = functools.partial(
    pl.kernel,
    out_type=(
        jax.ShapeDtypeStruct((NC, NPAD, D_IN), jnp.float32),
        jax.ShapeDtypeStruct((NW, NPAD), jnp.float32),
    ),
    mesh=_mesh,
    compiler_params=pltpu.CompilerParams(needs_layout_passes=False),
    scratch_types=[
        pltpu.VMEM((CHUNK,), jnp.int32),
        pltpu.VMEM((CHUNK,), jnp.int32),
        pltpu.VMEM((CHUNK, D_IN), jnp.float32),
        pltpu.VMEM((NPAD,), jnp.float32),
        pltpu.VMEM_SHARED((NPAD, D_IN), jnp.float32),
        pltpu.SemaphoreType.DMA,
    ],
)(_sc_agg1_body)


def _sc_agg2_body(src_hbm, dst_hbm, hlo_hbm, hhi_hbm, z2d_hbm, sums_hbm,
                  src_v, dst_v, rows_v, acc, sem):
  c = lax.axis_index("c")
  s = lax.axis_index("s")

  _zero_acc_slice(z2d_hbm, acc, s)
  plsc.subcore_barrier()

  edges_per_tile = N_EDGES // NS
  base0 = s * edges_per_tile
  n_chunks = edges_per_tile // CHUNK

  @pl.when(c == 0)
  def _():
    _agg_loop(src_hbm, dst_hbm, hlo_hbm, acc, src_v, dst_v, rows_v, sem,
              base0, n_chunks)

  @pl.when(c == 1)
  def _():
    _agg_loop(src_hbm, dst_hbm, hhi_hbm, acc, src_v, dst_v, rows_v, sem,
              base0, n_chunks)

  plsc.subcore_barrier()
  row0 = s * ROWS_PER_TILE
  pltpu.sync_copy(acc.at[pl.ds(row0, ROWS_PER_TILE), :],
                  sums_hbm.at[c, pl.ds(row0, ROWS_PER_TILE), :])


_sc_agg2 = functools.partial(
    pl.kernel,
    out_type=jax.ShapeDtypeStruct((NC, NPAD, D_IN), jnp.float32),
    mesh=_mesh,
    scratch_types=[
        pltpu.VMEM((CHUNK,), jnp.int32),
        pltpu.VMEM((CHUNK,), jnp.int32),
        pltpu.VMEM((CHUNK, D_IN), jnp.float32),
        pltpu.VMEM_SHARED((NPAD, D_IN), jnp.float32),
        pltpu.SemaphoreType.DMA,
    ],
)(_sc_agg2_body)


BLK = 1280


def _tc1_body(x_ref, sa_ref, sb_ref, degp_ref, w1rel_ref, b1_ref, w1root_ref,
              lo_ref, hi_ref):
  deg = jnp.sum(degp_ref[...], axis=0)
  inv = 1.0 / jnp.maximum(deg, 1.0)
  mean = (sa_ref[...][0] + sb_ref[...][0]) * inv[:, None]
  h = jnp.dot(mean, w1rel_ref[...], preferred_element_type=jnp.float32)
  h = h + jnp.dot(x_ref[...], w1root_ref[...], preferred_element_type=jnp.float32)
  h = h + b1_ref[...]
  h = jnp.maximum(h, 0.0)
  lo_ref[...] = h[:, :D_IN]
  hi_ref[...] = h[:, D_IN:]


def _tc2_body(mlo_ref, mhi_ref, degp_ref, hlo_ref, hhi_ref,
              w2rel_ref, b2_ref, w2root_ref, out_ref):
  deg = jnp.sum(degp_ref[...], axis=0)
  inv = 1.0 / jnp.maximum(deg, 1.0)
  mean = jnp.concatenate([mlo_ref[...][0], mhi_ref[...][0]], axis=1) * inv[:, None]
  h = jnp.concatenate([hlo_ref[...], hhi_ref[...]], axis=1)
  out = jnp.dot(mean, w2rel_ref[...], preferred_element_type=jnp.float32)
  out = out + jnp.dot(h, w2root_ref[...], preferred_element_type=jnp.float32)
  out_ref[...] = out + b2_ref[...]


def _tc1(xp, sums1, degp, W1_rel, b1, W1_root):
  grid = (NPAD // BLK,)
  return pl.pallas_call(
      _tc1_body,
      grid=grid,
      in_specs=[
          pl.BlockSpec((BLK, D_IN), lambda i: (i, 0)),
          pl.BlockSpec((1, BLK, D_IN), lambda i: (0, i, 0)),
          pl.BlockSpec((1, BLK, D_IN), lambda i: (1, i, 0)),
          pl.BlockSpec((NW, BLK), lambda i: (0, i)),
          pl.BlockSpec((D_IN, D_HID), lambda i: (0, 0)),
          pl.BlockSpec((1, D_HID), lambda i: (0, 0)),
          pl.BlockSpec((D_IN, D_HID), lambda i: (0, 0)),
      ],
      out_specs=[
          pl.BlockSpec((BLK, D_IN), lambda i: (i, 0)),
          pl.BlockSpec((BLK, D_IN), lambda i: (i, 0)),
      ],
      out_shape=[
          jax.ShapeDtypeStruct((NPAD, D_IN), jnp.float32),
          jax.ShapeDtypeStruct((NPAD, D_IN), jnp.float32),
      ],
  )(xp, sums1, sums1, degp, W1_rel, b1.reshape(1, D_HID), W1_root)


def _tc2(sums2, degp, h_lo, h_hi, W2_rel, b2, W2_root):
  grid = (NPAD // BLK,)
  return pl.pallas_call(
      _tc2_body,
      grid=grid,
      in_specs=[
          pl.BlockSpec((1, BLK, D_IN), lambda i: (0, i, 0)),
          pl.BlockSpec((1, BLK, D_IN), lambda i: (1, i, 0)),
          pl.BlockSpec((NW, BLK), lambda i: (0, i)),
          pl.BlockSpec((BLK, D_IN), lambda i: (i, 0)),
          pl.BlockSpec((BLK, D_IN), lambda i: (i, 0)),
          pl.BlockSpec((D_HID, D_HID), lambda i: (0, 0)),
          pl.BlockSpec((1, D_HID), lambda i: (0, 0)),
          pl.BlockSpec((D_HID, D_HID), lambda i: (0, 0)),
      ],
      out_specs=pl.BlockSpec((BLK, D_HID), lambda i: (i, 0)),
      out_shape=jax.ShapeDtypeStruct((NPAD, D_HID), jnp.float32),
  )(sums2, sums2, degp, h_lo, h_hi, W2_rel, b2.reshape(1, D_HID), W2_root)


def kernel(x, edge_index, W1_rel, b1, W1_root, W2_rel, b2, W2_root):
  src = edge_index[0].astype(jnp.int32)
  dst = edge_index[1].astype(jnp.int32)
  xp = jnp.pad(x, ((0, NPAD - N_NODES), (0, 0)))

  z2d = jnp.zeros((NPAD, D_IN), jnp.float32)
  z1d = jnp.zeros((NPAD,), jnp.float32)
  sums1, degp = _sc_agg1(src, dst, x, z2d, z1d)
  h_lo, h_hi = _tc1(xp, sums1, degp, W1_rel, b1, W1_root)
  sums2 = _sc_agg2(src, dst, h_lo, h_hi, z2d)
  out = _tc2(sums2, degp, h_lo, h_hi, W2_rel, b2, W2_root)
  return out[:N_NODES]


# speedup vs baseline: 6.7478x; 1.4750x over previous
import functools

import jax
import jax.numpy as jnp
from jax import lax
from jax.experimental import pallas as pl
from jax.experimental.pallas import tpu as pltpu
from jax.experimental.pallas import tpu_sc as plsc

N_NODES = 10000
N_EDGES = 320000
D_IN = 128
D_HID = 256

NC = 2
NS = 16
NW = NC * NS

NPAD = 10240
CHUNK = 40
CHUNKP = 48
G = 5
GB = G * CHUNK
NBLK = N_EDGES // GB
ROWS_PER_TILE = NPAD // NS

_mesh = plsc.VectorSubcoreMesh(core_axis_name="c", subcore_axis_name="s")


def _zero_acc_slice(z2d_hbm, acc, s):
  row0 = s * ROWS_PER_TILE
  pltpu.sync_copy(z2d_hbm.at[pl.ds(row0, ROWS_PER_TILE), :],
                  acc.at[pl.ds(row0, ROWS_PER_TILE), :])


def _agg_loop(eidx_hbm, tbl_hbm, acc, eidx_v, rows_v, gsem, ssem,
              blk0, n_blocks, deg_v=None):
  ones16 = jnp.ones((16,), jnp.float32)

  def block(b, carry):
    pltpu.sync_copy(eidx_hbm.at[b], eidx_v)
    for g in range(G):
      pltpu.async_copy(tbl_hbm.at[eidx_v.at[g, pl.ds(0, CHUNK)]],
                       rows_v.at[g, pl.ds(0, CHUNK), :], gsem.at[g])
    for g in range(G):
      pltpu.make_async_copy(tbl_hbm.at[eidx_v.at[g, pl.ds(0, CHUNK)]],
                            rows_v.at[g, pl.ds(0, CHUNK), :],
                            gsem.at[g]).wait()
      pltpu.sync_copy(rows_v.at[g], acc.at[eidx_v.at[G + g]], add=True)
      if deg_v is not None:
        for j in range(CHUNKP // 16):
          dvec = eidx_v[G + g, pl.ds(j * 16, 16)]
          plsc.addupdate_scatter(deg_v, [dvec], ones16)
    return carry

  lax.fori_loop(blk0, blk0 + n_blocks, block, 0)


def _sc_agg1_body(eidx_hbm, x_hbm, z2d_hbm, z1d_hbm, sums_hbm, degs_hbm,
                  eidx_v, rows_v, deg_v, acc, gsem, ssem):
  c = lax.axis_index("c")
  s = lax.axis_index("s")

  pltpu.sync_copy(z1d_hbm, deg_v)
  _zero_acc_slice(z2d_hbm, acc, s)
  plsc.subcore_barrier()

  blocks_per_tile = NBLK // NW
  blk0 = (c * NS + s) * blocks_per_tile
  _agg_loop(eidx_hbm, x_hbm, acc, eidx_v, rows_v, gsem, ssem,
            blk0, blocks_per_tile, deg_v=deg_v)

  plsc.subcore_barrier()
  row0 = s * ROWS_PER_TILE
  pltpu.sync_copy(acc.at[pl.ds(row0, ROWS_PER_TILE), :],
                  sums_hbm.at[c, pl.ds(row0, ROWS_PER_TILE), :])
  pltpu.sync_copy(deg_v, degs_hbm.at[c * NS + s])


_sc_agg1 = functools.partial(
    pl.kernel,
    out_type=(
        jax.ShapeDtypeStruct((NC, NPAD, D_IN), jnp.float32),
        jax.ShapeDtypeStruct((NW, NPAD), jnp.float32),
    ),
    mesh=_mesh,
    compiler_params=pltpu.CompilerParams(needs_layout_passes=False),
    scratch_types=[
        pltpu.VMEM((2 * G, CHUNKP), jnp.int32),
        pltpu.VMEM((G, CHUNKP, D_IN), jnp.float32),
        pltpu.VMEM((NPAD,), jnp.float32),
        pltpu.VMEM_SHARED((NPAD, D_IN), jnp.float32),
        pltpu.SemaphoreType.DMA((G,)),
        pltpu.SemaphoreType.DMA((G,)),
    ],
)(_sc_agg1_body)


def _sc_agg2_body(eidx_hbm, hlo_hbm, hhi_hbm, z2d_hbm, sums_hbm,
                  eidx_v, rows_v, acc, gsem, ssem):
  c = lax.axis_index("c")
  s = lax.axis_index("s")

  _zero_acc_slice(z2d_hbm, acc, s)
  plsc.subcore_barrier()

  blocks_per_tile = NBLK // NS
  blk0 = s * blocks_per_tile

  @pl.when(c == 0)
  def _():
    _agg_loop(eidx_hbm, hlo_hbm, acc, eidx_v, rows_v, gsem, ssem,
              blk0, blocks_per_tile)

  @pl.when(c == 1)
  def _():
    _agg_loop(eidx_hbm, hhi_hbm, acc, eidx_v, rows_v, gsem, ssem,
              blk0, blocks_per_tile)

  plsc.subcore_barrier()
  row0 = s * ROWS_PER_TILE
  pltpu.sync_copy(acc.at[pl.ds(row0, ROWS_PER_TILE), :],
                  sums_hbm.at[c, pl.ds(row0, ROWS_PER_TILE), :])


_sc_agg2 = functools.partial(
    pl.kernel,
    out_type=jax.ShapeDtypeStruct((NC, NPAD, D_IN), jnp.float32),
    mesh=_mesh,
    scratch_types=[
        pltpu.VMEM((2 * G, CHUNKP), jnp.int32),
        pltpu.VMEM((G, CHUNKP, D_IN), jnp.float32),
        pltpu.VMEM_SHARED((NPAD, D_IN), jnp.float32),
        pltpu.SemaphoreType.DMA((G,)),
        pltpu.SemaphoreType.DMA((G,)),
    ],
)(_sc_agg2_body)


BLK = 1280


def _tc1_body(x_ref, sa_ref, sb_ref, degp_ref, w1rel_ref, b1_ref, w1root_ref,
              lo_ref, hi_ref):
  deg = jnp.sum(degp_ref[...], axis=0)
  inv = 1.0 / jnp.maximum(deg, 1.0)
  mean = (sa_ref[...][0] + sb_ref[...][0]) * inv[:, None]
  h = jnp.dot(mean, w1rel_ref[...], preferred_element_type=jnp.float32)
  h = h + jnp.dot(x_ref[...], w1root_ref[...], preferred_element_type=jnp.float32)
  h = h + b1_ref[...]
  h = jnp.maximum(h, 0.0)
  lo_ref[...] = h[:, :D_IN]
  hi_ref[...] = h[:, D_IN:]


def _tc2_body(mlo_ref, mhi_ref, degp_ref, hlo_ref, hhi_ref,
              w2rel_ref, b2_ref, w2root_ref, out_ref):
  deg = jnp.sum(degp_ref[...], axis=0)
  inv = 1.0 / jnp.maximum(deg, 1.0)
  mean = jnp.concatenate([mlo_ref[...][0], mhi_ref[...][0]], axis=1) * inv[:, None]
  h = jnp.concatenate([hlo_ref[...], hhi_ref[...]], axis=1)
  out = jnp.dot(mean, w2rel_ref[...], preferred_element_type=jnp.float32)
  out = out + jnp.dot(h, w2root_ref[...], preferred_element_type=jnp.float32)
  out_ref[...] = out + b2_ref[...]


def _tc1(xp, sums1, degp, W1_rel, b1, W1_root):
  grid = (NPAD // BLK,)
  return pl.pallas_call(
      _tc1_body,
      grid=grid,
      in_specs=[
          pl.BlockSpec((BLK, D_IN), lambda i: (i, 0)),
          pl.BlockSpec((1, BLK, D_IN), lambda i: (0, i, 0)),
          pl.BlockSpec((1, BLK, D_IN), lambda i: (1, i, 0)),
          pl.BlockSpec((NW, BLK), lambda i: (0, i)),
          pl.BlockSpec((D_IN, D_HID), lambda i: (0, 0)),
          pl.BlockSpec((1, D_HID), lambda i: (0, 0)),
          pl.BlockSpec((D_IN, D_HID), lambda i: (0, 0)),
      ],
      out_specs=[
          pl.BlockSpec((BLK, D_IN), lambda i: (i, 0)),
          pl.BlockSpec((BLK, D_IN), lambda i: (i, 0)),
      ],
      out_shape=[
          jax.ShapeDtypeStruct((NPAD, D_IN), jnp.float32),
          jax.ShapeDtypeStruct((NPAD, D_IN), jnp.float32),
      ],
  )(xp, sums1, sums1, degp, W1_rel, b1.reshape(1, D_HID), W1_root)


def _tc2(sums2, degp, h_lo, h_hi, W2_rel, b2, W2_root):
  grid = (NPAD // BLK,)
  return pl.pallas_call(
      _tc2_body,
      grid=grid,
      in_specs=[
          pl.BlockSpec((1, BLK, D_IN), lambda i: (0, i, 0)),
          pl.BlockSpec((1, BLK, D_IN), lambda i: (1, i, 0)),
          pl.BlockSpec((NW, BLK), lambda i: (0, i)),
          pl.BlockSpec((BLK, D_IN), lambda i: (i, 0)),
          pl.BlockSpec((BLK, D_IN), lambda i: (i, 0)),
          pl.BlockSpec((D_HID, D_HID), lambda i: (0, 0)),
          pl.BlockSpec((1, D_HID), lambda i: (0, 0)),
          pl.BlockSpec((D_HID, D_HID), lambda i: (0, 0)),
      ],
      out_specs=pl.BlockSpec((BLK, D_HID), lambda i: (i, 0)),
      out_shape=jax.ShapeDtypeStruct((NPAD, D_HID), jnp.float32),
  )(sums2, sums2, degp, h_lo, h_hi, W2_rel, b2.reshape(1, D_HID), W2_root)


def kernel(x, edge_index, W1_rel, b1, W1_root, W2_rel, b2, W2_root):
  src = edge_index[0].astype(jnp.int32)
  dst = edge_index[1].astype(jnp.int32)
  srcp = jnp.pad(src.reshape(NBLK, G, CHUNK),
                 ((0, 0), (0, 0), (0, CHUNKP - CHUNK)))
  dstp = jnp.pad(dst.reshape(NBLK, G, CHUNK),
                 ((0, 0), (0, 0), (0, CHUNKP - CHUNK)),
                 constant_values=N_NODES)
  eidx = jnp.concatenate([srcp, dstp], axis=1)
  xp = jnp.pad(x, ((0, NPAD - N_NODES), (0, 0)))

  z2d = jnp.zeros((NPAD, D_IN), jnp.float32)
  z1d = jnp.zeros((NPAD,), jnp.float32)
  sums1, degp = _sc_agg1(eidx, x, z2d, z1d)
  h_lo, h_hi = _tc1(xp, sums1, degp, W1_rel, b1, W1_root)
  sums2 = _sc_agg2(eidx, h_lo, h_hi, z2d)
  out = _tc2(sums2, degp, h_lo, h_hi, W2_rel, b2, W2_root)
  return out[:N_NODES]


# speedup vs baseline: 7.5081x; 1.1127x over previous
import functools

import jax
import jax.numpy as jnp
from jax import lax
from jax.experimental import pallas as pl
from jax.experimental.pallas import tpu as pltpu
from jax.experimental.pallas import tpu_sc as plsc

N_NODES = 10000
N_EDGES = 320000
D_IN = 128
D_HID = 256

NC = 2
NS = 16
NW = NC * NS

NPAD = 10240
GB = 200
NBLK = N_EDGES // GB
ROWS_PER_TILE = NPAD // NS

_mesh = plsc.VectorSubcoreMesh(core_axis_name="c", subcore_axis_name="s")


def _zero_acc_slice(z2d_hbm, acc, s):
  row0 = s * ROWS_PER_TILE
  pltpu.sync_copy(z2d_hbm.at[pl.ds(row0, ROWS_PER_TILE), :],
                  acc.at[pl.ds(row0, ROWS_PER_TILE), :])


def _agg_loop(eidx_hbm, tbl_hbm, acc, eidx_v, rows_v, gsem,
              blk0, n_blocks, deg_v=None):
  ones16 = jnp.ones((16,), jnp.float32)
  GB2 = GB - 128

  def do_block(e, prefetch):
    E = eidx_v.at[e]
    g1 = pltpu.async_copy(tbl_hbm.at[E.at[0]], rows_v.at[pl.ds(0, 128), :],
                          gsem.at[0])
    g2 = pltpu.async_copy(tbl_hbm.at[E.at[1, pl.ds(0, GB2)]],
                          rows_v.at[pl.ds(128, GB2), :], gsem.at[1])
    prefetch()
    g1.wait()
    pltpu.sync_copy(rows_v.at[pl.ds(0, 128), :], acc.at[E.at[2]], add=True)
    g2.wait()
    pltpu.sync_copy(rows_v.at[pl.ds(128, 128), :], acc.at[E.at[3]], add=True)
    if deg_v is not None:
      for r in (2, 3):
        for j in range(128 // 16):
          dvec = eidx_v[e, r, pl.ds(j * 16, 16)]
          plsc.addupdate_scatter(deg_v, [dvec], ones16)

  pltpu.sync_copy(eidx_hbm.at[blk0], eidx_v.at[0])
  n_pairs = n_blocks // 2

  def pair(k, carry):
    b0 = blk0 + 2 * k
    do_block(0, lambda: pltpu.sync_copy(eidx_hbm.at[b0 + 1], eidx_v.at[1]))

    def prefetch_next():
      @pl.when(k + 1 < n_pairs)
      def _():
        pltpu.sync_copy(eidx_hbm.at[b0 + 2], eidx_v.at[0])

    do_block(1, prefetch_next)
    return carry

  lax.fori_loop(0, n_pairs, pair, 0)


def _sc_agg1_body(eidx_hbm, x_hbm, z2d_hbm, z1d_hbm, sums_hbm, degs_hbm,
                  eidx_v, rows_v, deg_v, acc, gsem):
  c = lax.axis_index("c")
  s = lax.axis_index("s")

  pltpu.sync_copy(z1d_hbm, deg_v)
  _zero_acc_slice(z2d_hbm, acc, s)
  plsc.subcore_barrier()

  blocks_per_tile = NBLK // NW
  blk0 = (c * NS + s) * blocks_per_tile
  _agg_loop(eidx_hbm, x_hbm, acc, eidx_v, rows_v, gsem,
            blk0, blocks_per_tile, deg_v=deg_v)

  plsc.subcore_barrier()
  row0 = s * ROWS_PER_TILE
  pltpu.sync_copy(acc.at[pl.ds(row0, ROWS_PER_TILE), :],
                  sums_hbm.at[c, pl.ds(row0, ROWS_PER_TILE), :])
  pltpu.sync_copy(deg_v, degs_hbm.at[c * NS + s])


_sc_agg1 = functools.partial(
    pl.kernel,
    out_type=(
        jax.ShapeDtypeStruct((NC, NPAD, D_IN), jnp.float32),
        jax.ShapeDtypeStruct((NW, NPAD), jnp.float32),
    ),
    mesh=_mesh,
    compiler_params=pltpu.CompilerParams(needs_layout_passes=False),
    scratch_types=[
        pltpu.VMEM((2, 4, 128), jnp.int32),
        pltpu.VMEM((256, D_IN), jnp.float32),
        pltpu.VMEM((NPAD,), jnp.float32),
        pltpu.VMEM_SHARED((NPAD, D_IN), jnp.float32),
        pltpu.SemaphoreType.DMA((2,)),
    ],
)(_sc_agg1_body)


def _sc_agg2_body(eidx_hbm, hlo_hbm, hhi_hbm, z2d_hbm, sums_hbm,
                  eidx_v, rows_v, acc, gsem):
  c = lax.axis_index("c")
  s = lax.axis_index("s")

  _zero_acc_slice(z2d_hbm, acc, s)
  plsc.subcore_barrier()

  blocks_per_tile = NBLK // NS
  blk0 = s * blocks_per_tile

  @pl.when(c == 0)
  def _():
    _agg_loop(eidx_hbm, hlo_hbm, acc, eidx_v, rows_v, gsem,
              blk0, blocks_per_tile)

  @pl.when(c == 1)
  def _():
    _agg_loop(eidx_hbm, hhi_hbm, acc, eidx_v, rows_v, gsem,
              blk0, blocks_per_tile)

  plsc.subcore_barrier()
  row0 = s * ROWS_PER_TILE
  pltpu.sync_copy(acc.at[pl.ds(row0, ROWS_PER_TILE), :],
                  sums_hbm.at[c, pl.ds(row0, ROWS_PER_TILE), :])


_sc_agg2 = functools.partial(
    pl.kernel,
    out_type=jax.ShapeDtypeStruct((NC, NPAD, D_IN), jnp.float32),
    mesh=_mesh,
    scratch_types=[
        pltpu.VMEM((2, 4, 128), jnp.int32),
        pltpu.VMEM((256, D_IN), jnp.float32),
        pltpu.VMEM_SHARED((NPAD, D_IN), jnp.float32),
        pltpu.SemaphoreType.DMA((2,)),
    ],
)(_sc_agg2_body)


BLK = 1280


def _tc1_body(x_ref, sa_ref, sb_ref, degp_ref, w1rel_ref, b1_ref, w1root_ref,
              lo_ref, hi_ref):
  deg = jnp.sum(degp_ref[...], axis=0)
  inv = 1.0 / jnp.maximum(deg, 1.0)
  mean = (sa_ref[...][0] + sb_ref[...][0]) * inv[:, None]
  h = jnp.dot(mean, w1rel_ref[...], preferred_element_type=jnp.float32)
  h = h + jnp.dot(x_ref[...], w1root_ref[...], preferred_element_type=jnp.float32)
  h = h + b1_ref[...]
  h = jnp.maximum(h, 0.0)
  lo_ref[...] = h[:, :D_IN]
  hi_ref[...] = h[:, D_IN:]


def _tc2_body(mlo_ref, mhi_ref, degp_ref, hlo_ref, hhi_ref,
              w2rel_ref, b2_ref, w2root_ref, out_ref):
  deg = jnp.sum(degp_ref[...], axis=0)
  inv = 1.0 / jnp.maximum(deg, 1.0)
  mean = jnp.concatenate([mlo_ref[...][0], mhi_ref[...][0]], axis=1) * inv[:, None]
  h = jnp.concatenate([hlo_ref[...], hhi_ref[...]], axis=1)
  out = jnp.dot(mean, w2rel_ref[...], preferred_element_type=jnp.float32)
  out = out + jnp.dot(h, w2root_ref[...], preferred_element_type=jnp.float32)
  out_ref[...] = out + b2_ref[...]


def _tc1(xp, sums1, degp, W1_rel, b1, W1_root):
  grid = (NPAD // BLK,)
  return pl.pallas_call(
      _tc1_body,
      grid=grid,
      in_specs=[
          pl.BlockSpec((BLK, D_IN), lambda i: (i, 0)),
          pl.BlockSpec((1, BLK, D_IN), lambda i: (0, i, 0)),
          pl.BlockSpec((1, BLK, D_IN), lambda i: (1, i, 0)),
          pl.BlockSpec((NW, BLK), lambda i: (0, i)),
          pl.BlockSpec((D_IN, D_HID), lambda i: (0, 0)),
          pl.BlockSpec((1, D_HID), lambda i: (0, 0)),
          pl.BlockSpec((D_IN, D_HID), lambda i: (0, 0)),
      ],
      out_specs=[
          pl.BlockSpec((BLK, D_IN), lambda i: (i, 0)),
          pl.BlockSpec((BLK, D_IN), lambda i: (i, 0)),
      ],
      out_shape=[
          jax.ShapeDtypeStruct((NPAD, D_IN), jnp.float32),
          jax.ShapeDtypeStruct((NPAD, D_IN), jnp.float32),
      ],
  )(xp, sums1, sums1, degp, W1_rel, b1.reshape(1, D_HID), W1_root)


def _tc2(sums2, degp, h_lo, h_hi, W2_rel, b2, W2_root):
  grid = (NPAD // BLK,)
  return pl.pallas_call(
      _tc2_body,
      grid=grid,
      in_specs=[
          pl.BlockSpec((1, BLK, D_IN), lambda i: (0, i, 0)),
          pl.BlockSpec((1, BLK, D_IN), lambda i: (1, i, 0)),
          pl.BlockSpec((NW, BLK), lambda i: (0, i)),
          pl.BlockSpec((BLK, D_IN), lambda i: (i, 0)),
          pl.BlockSpec((BLK, D_IN), lambda i: (i, 0)),
          pl.BlockSpec((D_HID, D_HID), lambda i: (0, 0)),
          pl.BlockSpec((1, D_HID), lambda i: (0, 0)),
          pl.BlockSpec((D_HID, D_HID), lambda i: (0, 0)),
      ],
      out_specs=pl.BlockSpec((BLK, D_HID), lambda i: (i, 0)),
      out_shape=jax.ShapeDtypeStruct((NPAD, D_HID), jnp.float32),
  )(sums2, sums2, degp, h_lo, h_hi, W2_rel, b2.reshape(1, D_HID), W2_root)


def kernel(x, edge_index, W1_rel, b1, W1_root, W2_rel, b2, W2_root):
  src = edge_index[0].astype(jnp.int32)
  dst = edge_index[1].astype(jnp.int32)
  srcb = src.reshape(NBLK, GB)
  dstb = dst.reshape(NBLK, GB)
  eidx = jnp.stack([
      srcb[:, :128],
      jnp.pad(srcb[:, 128:], ((0, 0), (0, 256 - GB))),
      dstb[:, :128],
      jnp.pad(dstb[:, 128:], ((0, 0), (0, 256 - GB)), constant_values=N_NODES),
  ], axis=1)
  xp = jnp.pad(x, ((0, NPAD - N_NODES), (0, 0)))

  z2d = jnp.zeros((NPAD, D_IN), jnp.float32)
  z1d = jnp.zeros((NPAD,), jnp.float32)
  sums1, degp = _sc_agg1(eidx, x, z2d, z1d)
  h_lo, h_hi = _tc1(xp, sums1, degp, W1_rel, b1, W1_root)
  sums2 = _sc_agg2(eidx, h_lo, h_hi, z2d)
  out = _tc2(sums2, degp, h_lo, h_hi, W2_rel, b2, W2_root)
  return out[:N_NODES]


# speedup vs baseline: 8.2717x; 1.1017x over previous
import functools

import jax
import jax.numpy as jnp
from jax import lax
from jax.experimental import pallas as pl
from jax.experimental.pallas import tpu as pltpu
from jax.experimental.pallas import tpu_sc as plsc

N_NODES = 10000
N_EDGES = 320000
D_IN = 128
D_HID = 256

NC = 2
NS = 16
NW = NC * NS

NPAD = 10240
GB = 200
NBLK = N_EDGES // GB
ROWS_PER_TILE = NPAD // NS

_mesh = plsc.VectorSubcoreMesh(core_axis_name="c", subcore_axis_name="s")


def _zero_acc_slice(z2d_hbm, acc, s):
  row0 = s * ROWS_PER_TILE
  pltpu.sync_copy(z2d_hbm.at[pl.ds(row0, ROWS_PER_TILE), :],
                  acc.at[pl.ds(row0, ROWS_PER_TILE), :])


W1 = 112
W2 = GB - W1


def _agg_loop(eidx_hbm, tbl_hbm, acc, eidx_v, rows_v, gsem, ssem,
              blk0, n_blocks, deg_v=None):
  ones16 = jnp.ones((16,), jnp.float32)

  def gathers(e):
    E = eidx_v.at[e]
    g0 = pltpu.async_copy(tbl_hbm.at[E.at[0]], rows_v.at[0], gsem.at[0])
    g1 = pltpu.async_copy(tbl_hbm.at[E.at[1, pl.ds(0, W2)]],
                          rows_v.at[1, pl.ds(0, W2), :], gsem.at[1])
    return g0, g1

  def scatters(e, g0, g1):
    E = eidx_v.at[e]
    g0.wait()
    s0 = pltpu.async_copy(rows_v.at[0], acc.at[E.at[2]], ssem.at[0],
                          add=True)
    g1.wait()
    s1 = pltpu.async_copy(rows_v.at[1], acc.at[E.at[3]], ssem.at[1],
                          add=True)
    if deg_v is not None:
      for r in (2, 3):
        for j in range(W1 // 16):
          dvec = eidx_v[e, r, pl.ds(j * 16, 16)]
          plsc.addupdate_scatter(deg_v, [dvec], ones16)
    return s0, s1

  pltpu.sync_copy(eidx_hbm.at[blk0], eidx_v.at[0])
  n_pairs = n_blocks // 2

  def pair(k, carry):
    b0 = blk0 + 2 * k
    g0, g1 = gathers(0)
    pltpu.sync_copy(eidx_hbm.at[b0 + 1], eidx_v.at[1])
    s0, s1 = scatters(0, g0, g1)
    s0.wait()
    s1.wait()
    g0, g1 = gathers(1)

    @pl.when(k + 1 < n_pairs)
    def _():
      pltpu.sync_copy(eidx_hbm.at[b0 + 2], eidx_v.at[0])

    s0, s1 = scatters(1, g0, g1)
    s0.wait()
    s1.wait()
    return carry

  lax.fori_loop(0, n_pairs, pair, 0)


def _sc_agg1_body(eidx_hbm, x_hbm, z2d_hbm, z1d_hbm, sums_hbm, degs_hbm,
                  eidx_v, rows_v, deg_v, acc, gsem, ssem):
  c = lax.axis_index("c")
  s = lax.axis_index("s")

  pltpu.sync_copy(z1d_hbm, deg_v)
  _zero_acc_slice(z2d_hbm, acc, s)
  plsc.subcore_barrier()

  blocks_per_tile = NBLK // NW
  blk0 = (c * NS + s) * blocks_per_tile
  _agg_loop(eidx_hbm, x_hbm, acc, eidx_v, rows_v, gsem, ssem,
            blk0, blocks_per_tile, deg_v=deg_v)

  plsc.subcore_barrier()
  row0 = s * ROWS_PER_TILE
  pltpu.sync_copy(acc.at[pl.ds(row0, ROWS_PER_TILE), :],
                  sums_hbm.at[c, pl.ds(row0, ROWS_PER_TILE), :])
  pltpu.sync_copy(deg_v, degs_hbm.at[c * NS + s])


_sc_agg1 = functools.partial(
    pl.kernel,
    out_type=(
        jax.ShapeDtypeStruct((NC, NPAD, D_IN), jnp.float32),
        jax.ShapeDtypeStruct((NW, NPAD), jnp.float32),
    ),
    mesh=_mesh,
    compiler_params=pltpu.CompilerParams(needs_layout_passes=False),
    scratch_types=[
        pltpu.VMEM((2, 4, W1), jnp.int32),
        pltpu.VMEM((2, W1, D_IN), jnp.float32),
        pltpu.VMEM((NPAD,), jnp.float32),
        pltpu.VMEM_SHARED((NPAD, D_IN), jnp.float32),
        pltpu.SemaphoreType.DMA((2,)),
        pltpu.SemaphoreType.DMA((2,)),
    ],
)(_sc_agg1_body)


def _sc_agg2_body(eidx_hbm, hlo_hbm, hhi_hbm, z2d_hbm, sums_hbm,
                  eidx_v, rows_v, acc, gsem, ssem):
  c = lax.axis_index("c")
  s = lax.axis_index("s")

  _zero_acc_slice(z2d_hbm, acc, s)
  plsc.subcore_barrier()

  blocks_per_tile = NBLK // NS
  blk0 = s * blocks_per_tile

  @pl.when(c == 0)
  def _():
    _agg_loop(eidx_hbm, hlo_hbm, acc, eidx_v, rows_v, gsem, ssem,
              blk0, blocks_per_tile)

  @pl.when(c == 1)
  def _():
    _agg_loop(eidx_hbm, hhi_hbm, acc, eidx_v, rows_v, gsem, ssem,
              blk0, blocks_per_tile)

  plsc.subcore_barrier()
  row0 = s * ROWS_PER_TILE
  pltpu.sync_copy(acc.at[pl.ds(row0, ROWS_PER_TILE), :],
                  sums_hbm.at[c, pl.ds(row0, ROWS_PER_TILE), :])


_sc_agg2 = functools.partial(
    pl.kernel,
    out_type=jax.ShapeDtypeStruct((NC, NPAD, D_IN), jnp.float32),
    mesh=_mesh,
    scratch_types=[
        pltpu.VMEM((2, 4, W1), jnp.int32),
        pltpu.VMEM((2, W1, D_IN), jnp.float32),
        pltpu.VMEM_SHARED((NPAD, D_IN), jnp.float32),
        pltpu.SemaphoreType.DMA((2,)),
        pltpu.SemaphoreType.DMA((2,)),
    ],
)(_sc_agg2_body)


BLK = 1280


def _tc1_body(x_ref, sa_ref, sb_ref, degp_ref, w1rel_ref, b1_ref, w1root_ref,
              lo_ref, hi_ref):
  deg = jnp.sum(degp_ref[...], axis=0)
  inv = 1.0 / jnp.maximum(deg, 1.0)
  mean = (sa_ref[...][0] + sb_ref[...][0]) * inv[:, None]
  h = jnp.dot(mean, w1rel_ref[...], preferred_element_type=jnp.float32)
  h = h + jnp.dot(x_ref[...], w1root_ref[...], preferred_element_type=jnp.float32)
  h = h + b1_ref[...]
  h = jnp.maximum(h, 0.0)
  lo_ref[...] = h[:, :D_IN]
  hi_ref[...] = h[:, D_IN:]


def _tc2_body(mlo_ref, mhi_ref, degp_ref, hlo_ref, hhi_ref,
              w2rel_ref, b2_ref, w2root_ref, out_ref):
  deg = jnp.sum(degp_ref[...], axis=0)
  inv = 1.0 / jnp.maximum(deg, 1.0)
  mean = jnp.concatenate([mlo_ref[...][0], mhi_ref[...][0]], axis=1) * inv[:, None]
  h = jnp.concatenate([hlo_ref[...], hhi_ref[...]], axis=1)
  out = jnp.dot(mean, w2rel_ref[...], preferred_element_type=jnp.float32)
  out = out + jnp.dot(h, w2root_ref[...], preferred_element_type=jnp.float32)
  out_ref[...] = out + b2_ref[...]


def _tc1(xp, sums1, degp, W1_rel, b1, W1_root):
  grid = (NPAD // BLK,)
  return pl.pallas_call(
      _tc1_body,
      grid=grid,
      in_specs=[
          pl.BlockSpec((BLK, D_IN), lambda i: (i, 0)),
          pl.BlockSpec((1, BLK, D_IN), lambda i: (0, i, 0)),
          pl.BlockSpec((1, BLK, D_IN), lambda i: (1, i, 0)),
          pl.BlockSpec((NW, BLK), lambda i: (0, i)),
          pl.BlockSpec((D_IN, D_HID), lambda i: (0, 0)),
          pl.BlockSpec((1, D_HID), lambda i: (0, 0)),
          pl.BlockSpec((D_IN, D_HID), lambda i: (0, 0)),
      ],
      out_specs=[
          pl.BlockSpec((BLK, D_IN), lambda i: (i, 0)),
          pl.BlockSpec((BLK, D_IN), lambda i: (i, 0)),
      ],
      out_shape=[
          jax.ShapeDtypeStruct((NPAD, D_IN), jnp.float32),
          jax.ShapeDtypeStruct((NPAD, D_IN), jnp.float32),
      ],
  )(xp, sums1, sums1, degp, W1_rel, b1.reshape(1, D_HID), W1_root)


def _tc2(sums2, degp, h_lo, h_hi, W2_rel, b2, W2_root):
  grid = (NPAD // BLK,)
  return pl.pallas_call(
      _tc2_body,
      grid=grid,
      in_specs=[
          pl.BlockSpec((1, BLK, D_IN), lambda i: (0, i, 0)),
          pl.BlockSpec((1, BLK, D_IN), lambda i: (1, i, 0)),
          pl.BlockSpec((NW, BLK), lambda i: (0, i)),
          pl.BlockSpec((BLK, D_IN), lambda i: (i, 0)),
          pl.BlockSpec((BLK, D_IN), lambda i: (i, 0)),
          pl.BlockSpec((D_HID, D_HID), lambda i: (0, 0)),
          pl.BlockSpec((1, D_HID), lambda i: (0, 0)),
          pl.BlockSpec((D_HID, D_HID), lambda i: (0, 0)),
      ],
      out_specs=pl.BlockSpec((BLK, D_HID), lambda i: (i, 0)),
      out_shape=jax.ShapeDtypeStruct((NPAD, D_HID), jnp.float32),
  )(sums2, sums2, degp, h_lo, h_hi, W2_rel, b2.reshape(1, D_HID), W2_root)


def kernel(x, edge_index, W1_rel, b1, W1_root, W2_rel, b2, W2_root):
  src = edge_index[0].astype(jnp.int32)
  dst = edge_index[1].astype(jnp.int32)
  srcb = src.reshape(NBLK, GB)
  dstb = dst.reshape(NBLK, GB)
  eidx = jnp.stack([
      srcb[:, :W1],
      jnp.pad(srcb[:, W1:], ((0, 0), (0, 2 * W1 - GB))),
      dstb[:, :W1],
      jnp.pad(dstb[:, W1:], ((0, 0), (0, 2 * W1 - GB)), constant_values=N_NODES),
  ], axis=1)
  xp = jnp.pad(x, ((0, NPAD - N_NODES), (0, 0)))

  z2d = jnp.zeros((NPAD, D_IN), jnp.float32)
  z1d = jnp.zeros((NPAD,), jnp.float32)
  sums1, degp = _sc_agg1(eidx, x, z2d, z1d)
  h_lo, h_hi = _tc1(xp, sums1, degp, W1_rel, b1, W1_root)
  sums2 = _sc_agg2(eidx, h_lo, h_hi, z2d)
  out = _tc2(sums2, degp, h_lo, h_hi, W2_rel, b2, W2_root)
  return out[:N_NODES]


# speedup vs baseline: 8.3158x; 1.0053x over previous
import functools

import jax
import jax.numpy as jnp
from jax import lax
from jax.experimental import pallas as pl
from jax.experimental.pallas import tpu as pltpu
from jax.experimental.pallas import tpu_sc as plsc

N_NODES = 10000
N_EDGES = 320000
D_IN = 128
D_HID = 256

NC = 2
NS = 16
NW = NC * NS

NPAD = 10240
GB = 200
NBLK = N_EDGES // GB
ROWS_PER_TILE = NPAD // NS

_mesh = plsc.VectorSubcoreMesh(core_axis_name="c", subcore_axis_name="s")


def _zero_acc_slice(z2d_hbm, acc, s):
  row0 = s * ROWS_PER_TILE
  pltpu.sync_copy(z2d_hbm.at[pl.ds(row0, ROWS_PER_TILE), :],
                  acc.at[pl.ds(row0, ROWS_PER_TILE), :])


W1 = 112
W2 = GB - W1


def _agg_loop(eidx_hbm, tbl_hbm, acc, eidx_v, rows_v, gsem, ssem,
              blk0, n_blocks, deg_v=None):
  ones16 = jnp.ones((16,), jnp.float32)

  def gathers(e):
    E = eidx_v.at[e]
    g0 = pltpu.async_copy(tbl_hbm.at[E.at[0]], rows_v.at[0], gsem.at[0])
    g1 = pltpu.async_copy(tbl_hbm.at[E.at[1, pl.ds(0, W2)]],
                          rows_v.at[1, pl.ds(0, W2), :], gsem.at[1])
    return g0, g1

  def scatters(e, g0, g1):
    E = eidx_v.at[e]
    g0.wait()
    s0 = pltpu.async_copy(rows_v.at[0], acc.at[E.at[2]], ssem.at[0],
                          add=True)
    g1.wait()
    s1 = pltpu.async_copy(rows_v.at[1], acc.at[E.at[3]], ssem.at[1],
                          add=True)
    if deg_v is not None:
      for r in (2, 3):
        for j in range(W1 // 16):
          dvec = eidx_v[e, r, pl.ds(j * 16, 16)]
          plsc.addupdate_scatter(deg_v, [dvec], ones16)
    return s0, s1

  pltpu.sync_copy(eidx_hbm.at[blk0], eidx_v.at[0])
  n_pairs = n_blocks // 2

  def pair(k, carry):
    b0 = blk0 + 2 * k
    g0, g1 = gathers(0)
    pltpu.sync_copy(eidx_hbm.at[b0 + 1], eidx_v.at[1])
    s0, s1 = scatters(0, g0, g1)
    s0.wait()
    E1 = eidx_v.at[1]
    g0 = pltpu.async_copy(tbl_hbm.at[E1.at[0]], rows_v.at[0], gsem.at[0])
    s1.wait()
    g1 = pltpu.async_copy(tbl_hbm.at[E1.at[1, pl.ds(0, W2)]],
                          rows_v.at[1, pl.ds(0, W2), :], gsem.at[1])

    @pl.when(k + 1 < n_pairs)
    def _():
      pltpu.sync_copy(eidx_hbm.at[b0 + 2], eidx_v.at[0])

    s0, s1 = scatters(1, g0, g1)
    s0.wait()
    s1.wait()
    return carry

  lax.fori_loop(0, n_pairs, pair, 0)


def _sc_agg1_body(eidx_hbm, x_hbm, z2d_hbm, z1d_hbm, sums_hbm, degs_hbm,
                  eidx_v, rows_v, deg_v, acc, gsem, ssem):
  c = lax.axis_index("c")
  s = lax.axis_index("s")

  pltpu.sync_copy(z1d_hbm, deg_v)
  _zero_acc_slice(z2d_hbm, acc, s)
  plsc.subcore_barrier()

  blocks_per_tile = NBLK // NW
  blk0 = (c * NS + s) * blocks_per_tile
  _agg_loop(eidx_hbm, x_hbm, acc, eidx_v, rows_v, gsem, ssem,
            blk0, blocks_per_tile, deg_v=deg_v)

  plsc.subcore_barrier()
  row0 = s * ROWS_PER_TILE
  pltpu.sync_copy(acc.at[pl.ds(row0, ROWS_PER_TILE), :],
                  sums_hbm.at[c, pl.ds(row0, ROWS_PER_TILE), :])
  pltpu.sync_copy(deg_v, degs_hbm.at[c * NS + s])


_sc_agg1 = functools.partial(
    pl.kernel,
    out_type=(
        jax.ShapeDtypeStruct((NC, NPAD, D_IN), jnp.float32),
        jax.ShapeDtypeStruct((NW, NPAD), jnp.float32),
    ),
    mesh=_mesh,
    compiler_params=pltpu.CompilerParams(needs_layout_passes=False),
    scratch_types=[
        pltpu.VMEM((2, 4, W1), jnp.int32),
        pltpu.VMEM((2, W1, D_IN), jnp.float32),
        pltpu.VMEM((NPAD,), jnp.float32),
        pltpu.VMEM_SHARED((NPAD, D_IN), jnp.float32),
        pltpu.SemaphoreType.DMA((2,)),
        pltpu.SemaphoreType.DMA((2,)),
    ],
)(_sc_agg1_body)


def _sc_agg2_body(eidx_hbm, hlo_hbm, hhi_hbm, z2d_hbm, sums_hbm,
                  eidx_v, rows_v, acc, gsem, ssem):
  c = lax.axis_index("c")
  s = lax.axis_index("s")

  _zero_acc_slice(z2d_hbm, acc, s)
  plsc.subcore_barrier()

  blocks_per_tile = NBLK // NS
  blk0 = s * blocks_per_tile

  @pl.when(c == 0)
  def _():
    _agg_loop(eidx_hbm, hlo_hbm, acc, eidx_v, rows_v, gsem, ssem,
              blk0, blocks_per_tile)

  @pl.when(c == 1)
  def _():
    _agg_loop(eidx_hbm, hhi_hbm, acc, eidx_v, rows_v, gsem, ssem,
              blk0, blocks_per_tile)

  plsc.subcore_barrier()
  row0 = s * ROWS_PER_TILE
  pltpu.sync_copy(acc.at[pl.ds(row0, ROWS_PER_TILE), :],
                  sums_hbm.at[c, pl.ds(row0, ROWS_PER_TILE), :])


_sc_agg2 = functools.partial(
    pl.kernel,
    out_type=jax.ShapeDtypeStruct((NC, NPAD, D_IN), jnp.float32),
    mesh=_mesh,
    scratch_types=[
        pltpu.VMEM((2, 4, W1), jnp.int32),
        pltpu.VMEM((2, W1, D_IN), jnp.float32),
        pltpu.VMEM_SHARED((NPAD, D_IN), jnp.float32),
        pltpu.SemaphoreType.DMA((2,)),
        pltpu.SemaphoreType.DMA((2,)),
    ],
)(_sc_agg2_body)


BLK = 1280


def _tc1_body(x_ref, sa_ref, sb_ref, degp_ref, w1rel_ref, b1_ref, w1root_ref,
              lo_ref, hi_ref):
  deg = jnp.sum(degp_ref[...], axis=0)
  inv = 1.0 / jnp.maximum(deg, 1.0)
  mean = (sa_ref[...][0] + sb_ref[...][0]) * inv[:, None]
  h = jnp.dot(mean, w1rel_ref[...], preferred_element_type=jnp.float32)
  h = h + jnp.dot(x_ref[...], w1root_ref[...], preferred_element_type=jnp.float32)
  h = h + b1_ref[...]
  h = jnp.maximum(h, 0.0)
  lo_ref[...] = h[:, :D_IN]
  hi_ref[...] = h[:, D_IN:]


def _tc2_body(mlo_ref, mhi_ref, degp_ref, hlo_ref, hhi_ref,
              w2rel_ref, b2_ref, w2root_ref, out_ref):
  deg = jnp.sum(degp_ref[...], axis=0)
  inv = 1.0 / jnp.maximum(deg, 1.0)
  mean = jnp.concatenate([mlo_ref[...][0], mhi_ref[...][0]], axis=1) * inv[:, None]
  h = jnp.concatenate([hlo_ref[...], hhi_ref[...]], axis=1)
  out = jnp.dot(mean, w2rel_ref[...], preferred_element_type=jnp.float32)
  out = out + jnp.dot(h, w2root_ref[...], preferred_element_type=jnp.float32)
  out_ref[...] = out + b2_ref[...]


def _tc1(xp, sums1, degp, W1_rel, b1, W1_root):
  grid = (NPAD // BLK,)
  return pl.pallas_call(
      _tc1_body,
      grid=grid,
      in_specs=[
          pl.BlockSpec((BLK, D_IN), lambda i: (i, 0)),
          pl.BlockSpec((1, BLK, D_IN), lambda i: (0, i, 0)),
          pl.BlockSpec((1, BLK, D_IN), lambda i: (1, i, 0)),
          pl.BlockSpec((NW, BLK), lambda i: (0, i)),
          pl.BlockSpec((D_IN, D_HID), lambda i: (0, 0)),
          pl.BlockSpec((1, D_HID), lambda i: (0, 0)),
          pl.BlockSpec((D_IN, D_HID), lambda i: (0, 0)),
      ],
      out_specs=[
          pl.BlockSpec((BLK, D_IN), lambda i: (i, 0)),
          pl.BlockSpec((BLK, D_IN), lambda i: (i, 0)),
      ],
      out_shape=[
          jax.ShapeDtypeStruct((NPAD, D_IN), jnp.float32),
          jax.ShapeDtypeStruct((NPAD, D_IN), jnp.float32),
      ],
  )(xp, sums1, sums1, degp, W1_rel, b1.reshape(1, D_HID), W1_root)


def _tc2(sums2, degp, h_lo, h_hi, W2_rel, b2, W2_root):
  grid = (NPAD // BLK,)
  return pl.pallas_call(
      _tc2_body,
      grid=grid,
      in_specs=[
          pl.BlockSpec((1, BLK, D_IN), lambda i: (0, i, 0)),
          pl.BlockSpec((1, BLK, D_IN), lambda i: (1, i, 0)),
          pl.BlockSpec((NW, BLK), lambda i: (0, i)),
          pl.BlockSpec((BLK, D_IN), lambda i: (i, 0)),
          pl.BlockSpec((BLK, D_IN), lambda i: (i, 0)),
          pl.BlockSpec((D_HID, D_HID), lambda i: (0, 0)),
          pl.BlockSpec((1, D_HID), lambda i: (0, 0)),
          pl.BlockSpec((D_HID, D_HID), lambda i: (0, 0)),
      ],
      out_specs=pl.BlockSpec((BLK, D_HID), lambda i: (i, 0)),
      out_shape=jax.ShapeDtypeStruct((NPAD, D_HID), jnp.float32),
  )(sums2, sums2, degp, h_lo, h_hi, W2_rel, b2.reshape(1, D_HID), W2_root)


def kernel(x, edge_index, W1_rel, b1, W1_root, W2_rel, b2, W2_root):
  src = edge_index[0].astype(jnp.int32)
  dst = edge_index[1].astype(jnp.int32)
  srcb = src.reshape(NBLK, GB)
  dstb = dst.reshape(NBLK, GB)
  eidx = jnp.stack([
      srcb[:, :W1],
      jnp.pad(srcb[:, W1:], ((0, 0), (0, 2 * W1 - GB))),
      dstb[:, :W1],
      jnp.pad(dstb[:, W1:], ((0, 0), (0, 2 * W1 - GB)), constant_values=N_NODES),
  ], axis=1)
  xp = jnp.pad(x, ((0, NPAD - N_NODES), (0, 0)))

  z2d = jnp.zeros((NPAD, D_IN), jnp.float32)
  z1d = jnp.zeros((NPAD,), jnp.float32)
  sums1, degp = _sc_agg1(eidx, x, z2d, z1d)
  h_lo, h_hi = _tc1(xp, sums1, degp, W1_rel, b1, W1_root)
  sums2 = _sc_agg2(eidx, h_lo, h_hi, z2d)
  out = _tc2(sums2, degp, h_lo, h_hi, W2_rel, b2, W2_root)
  return out[:N_NODES]


# speedup vs baseline: 8.3535x; 1.0045x over previous
import functools

import jax
import jax.numpy as jnp
from jax import lax
from jax.experimental import pallas as pl
from jax.experimental.pallas import tpu as pltpu
from jax.experimental.pallas import tpu_sc as plsc

N_NODES = 10000
N_EDGES = 320000
D_IN = 128
D_HID = 256

NC = 2
NS = 16
NW = NC * NS

NPAD = 10240
GB = 256
NBLK = N_EDGES // GB
ROWS_PER_TILE = NPAD // NS

_mesh = plsc.VectorSubcoreMesh(core_axis_name="c", subcore_axis_name="s")


def _zero_acc_slice(z2d_hbm, acc, s):
  row0 = s * ROWS_PER_TILE
  pltpu.sync_copy(z2d_hbm.at[pl.ds(row0, ROWS_PER_TILE), :],
                  acc.at[pl.ds(row0, ROWS_PER_TILE), :])


W = 128


def _agg_loop(eidx_hbm, tbl_hbm, acc, eidx_v, rows_v, gsem, ssem,
              blk0, blk_end, deg_v=None):
  ones16 = jnp.ones((16,), jnp.float32)

  def block(b, carry):
    pltpu.sync_copy(eidx_hbm.at[b], eidx_v)
    g0 = pltpu.async_copy(tbl_hbm.at[eidx_v.at[0]], rows_v.at[0], gsem.at[0])
    g1 = pltpu.async_copy(tbl_hbm.at[eidx_v.at[1]], rows_v.at[1], gsem.at[1])
    g0.wait()
    s0 = pltpu.async_copy(rows_v.at[0], acc.at[eidx_v.at[2]], ssem.at[0],
                          add=True)
    g1.wait()
    s1 = pltpu.async_copy(rows_v.at[1], acc.at[eidx_v.at[3]], ssem.at[1],
                          add=True)
    if deg_v is not None:
      for r in (2, 3):
        for j in range(W // 16):
          dvec = eidx_v[r, pl.ds(j * 16, 16)]
          plsc.addupdate_scatter(deg_v, [dvec], ones16)
    s0.wait()
    s1.wait()
    return carry

  lax.fori_loop(blk0, blk_end, block, 0)


def _sc_agg1_body(eidx_hbm, x_hbm, z2d_hbm, z1d_hbm, sums_hbm, degs_hbm,
                  eidx_v, rows_v, deg_v, acc, gsem, ssem):
  c = lax.axis_index("c")
  s = lax.axis_index("s")

  pltpu.sync_copy(z1d_hbm, deg_v)
  _zero_acc_slice(z2d_hbm, acc, s)
  plsc.subcore_barrier()

  w = c * NS + s
  _agg_loop(eidx_hbm, x_hbm, acc, eidx_v, rows_v, gsem, ssem,
            w * NBLK // NW, (w + 1) * NBLK // NW, deg_v=deg_v)

  plsc.subcore_barrier()
  row0 = s * ROWS_PER_TILE
  pltpu.sync_copy(acc.at[pl.ds(row0, ROWS_PER_TILE), :],
                  sums_hbm.at[c, pl.ds(row0, ROWS_PER_TILE), :])
  pltpu.sync_copy(deg_v, degs_hbm.at[c * NS + s])


_sc_agg1 = functools.partial(
    pl.kernel,
    out_type=(
        jax.ShapeDtypeStruct((NC, NPAD, D_IN), jnp.float32),
        jax.ShapeDtypeStruct((NW, NPAD), jnp.float32),
    ),
    mesh=_mesh,
    compiler_params=pltpu.CompilerParams(needs_layout_passes=False),
    scratch_types=[
        pltpu.VMEM((4, W), jnp.int32),
        pltpu.VMEM((2, W, D_IN), jnp.float32),
        pltpu.VMEM((NPAD,), jnp.float32),
        pltpu.VMEM_SHARED((NPAD, D_IN), jnp.float32),
        pltpu.SemaphoreType.DMA((2,)),
        pltpu.SemaphoreType.DMA((2,)),
    ],
)(_sc_agg1_body)


def _sc_agg2_body(eidx_hbm, hlo_hbm, hhi_hbm, z2d_hbm, sums_hbm,
                  eidx_v, rows_v, acc, gsem, ssem):
  c = lax.axis_index("c")
  s = lax.axis_index("s")

  _zero_acc_slice(z2d_hbm, acc, s)
  plsc.subcore_barrier()

  blk0 = s * NBLK // NS
  blk_end = (s + 1) * NBLK // NS

  @pl.when(c == 0)
  def _():
    _agg_loop(eidx_hbm, hlo_hbm, acc, eidx_v, rows_v, gsem, ssem,
              blk0, blk_end)

  @pl.when(c == 1)
  def _():
    _agg_loop(eidx_hbm, hhi_hbm, acc, eidx_v, rows_v, gsem, ssem,
              blk0, blk_end)

  plsc.subcore_barrier()
  row0 = s * ROWS_PER_TILE
  pltpu.sync_copy(acc.at[pl.ds(row0, ROWS_PER_TILE), :],
                  sums_hbm.at[c, pl.ds(row0, ROWS_PER_TILE), :])


_sc_agg2 = functools.partial(
    pl.kernel,
    out_type=jax.ShapeDtypeStruct((NC, NPAD, D_IN), jnp.float32),
    mesh=_mesh,
    scratch_types=[
        pltpu.VMEM((4, W), jnp.int32),
        pltpu.VMEM((2, W, D_IN), jnp.float32),
        pltpu.VMEM_SHARED((NPAD, D_IN), jnp.float32),
        pltpu.SemaphoreType.DMA((2,)),
        pltpu.SemaphoreType.DMA((2,)),
    ],
)(_sc_agg2_body)


BLK = 1280


def _tc1_body(x_ref, sa_ref, sb_ref, degp_ref, w1rel_ref, b1_ref, w1root_ref,
              lo_ref, hi_ref):
  deg = jnp.sum(degp_ref[...], axis=0)
  inv = 1.0 / jnp.maximum(deg, 1.0)
  mean = (sa_ref[...][0] + sb_ref[...][0]) * inv[:, None]
  h = jnp.dot(mean, w1rel_ref[...], preferred_element_type=jnp.float32)
  h = h + jnp.dot(x_ref[...], w1root_ref[...], preferred_element_type=jnp.float32)
  h = h + b1_ref[...]
  h = jnp.maximum(h, 0.0)
  lo_ref[...] = h[:, :D_IN]
  hi_ref[...] = h[:, D_IN:]


def _tc2_body(mlo_ref, mhi_ref, degp_ref, hlo_ref, hhi_ref,
              w2rel_ref, b2_ref, w2root_ref, out_ref):
  deg = jnp.sum(degp_ref[...], axis=0)
  inv = 1.0 / jnp.maximum(deg, 1.0)
  mean = jnp.concatenate([mlo_ref[...][0], mhi_ref[...][0]], axis=1) * inv[:, None]
  h = jnp.concatenate([hlo_ref[...], hhi_ref[...]], axis=1)
  out = jnp.dot(mean, w2rel_ref[...], preferred_element_type=jnp.float32)
  out = out + jnp.dot(h, w2root_ref[...], preferred_element_type=jnp.float32)
  out_ref[...] = out + b2_ref[...]


def _tc1(xp, sums1, degp, W1_rel, b1, W1_root):
  grid = (NPAD // BLK,)
  return pl.pallas_call(
      _tc1_body,
      grid=grid,
      in_specs=[
          pl.BlockSpec((BLK, D_IN), lambda i: (i, 0)),
          pl.BlockSpec((1, BLK, D_IN), lambda i: (0, i, 0)),
          pl.BlockSpec((1, BLK, D_IN), lambda i: (1, i, 0)),
          pl.BlockSpec((NW, BLK), lambda i: (0, i)),
          pl.BlockSpec((D_IN, D_HID), lambda i: (0, 0)),
          pl.BlockSpec((1, D_HID), lambda i: (0, 0)),
          pl.BlockSpec((D_IN, D_HID), lambda i: (0, 0)),
      ],
      out_specs=[
          pl.BlockSpec((BLK, D_IN), lambda i: (i, 0)),
          pl.BlockSpec((BLK, D_IN), lambda i: (i, 0)),
      ],
      out_shape=[
          jax.ShapeDtypeStruct((NPAD, D_IN), jnp.float32),
          jax.ShapeDtypeStruct((NPAD, D_IN), jnp.float32),
      ],
  )(xp, sums1, sums1, degp, W1_rel, b1.reshape(1, D_HID), W1_root)


def _tc2(sums2, degp, h_lo, h_hi, W2_rel, b2, W2_root):
  grid = (NPAD // BLK,)
  return pl.pallas_call(
      _tc2_body,
      grid=grid,
      in_specs=[
          pl.BlockSpec((1, BLK, D_IN), lambda i: (0, i, 0)),
          pl.BlockSpec((1, BLK, D_IN), lambda i: (1, i, 0)),
          pl.BlockSpec((NW, BLK), lambda i: (0, i)),
          pl.BlockSpec((BLK, D_IN), lambda i: (i, 0)),
          pl.BlockSpec((BLK, D_IN), lambda i: (i, 0)),
          pl.BlockSpec((D_HID, D_HID), lambda i: (0, 0)),
          pl.BlockSpec((1, D_HID), lambda i: (0, 0)),
          pl.BlockSpec((D_HID, D_HID), lambda i: (0, 0)),
      ],
      out_specs=pl.BlockSpec((BLK, D_HID), lambda i: (i, 0)),
      out_shape=jax.ShapeDtypeStruct((NPAD, D_HID), jnp.float32),
  )(sums2, sums2, degp, h_lo, h_hi, W2_rel, b2.reshape(1, D_HID), W2_root)


def kernel(x, edge_index, W1_rel, b1, W1_root, W2_rel, b2, W2_root):
  src = edge_index[0].astype(jnp.int32)
  dst = edge_index[1].astype(jnp.int32)
  srcb = src.reshape(NBLK, 2, W)
  dstb = dst.reshape(NBLK, 2, W)
  eidx = jnp.concatenate([srcb, dstb], axis=1)
  xp = jnp.pad(x, ((0, NPAD - N_NODES), (0, 0)))

  z2d = jnp.zeros((NPAD, D_IN), jnp.float32)
  z1d = jnp.zeros((NPAD,), jnp.float32)
  sums1, degp = _sc_agg1(eidx, x, z2d, z1d)
  h_lo, h_hi = _tc1(xp, sums1, degp, W1_rel, b1, W1_root)
  sums2 = _sc_agg2(eidx, h_lo, h_hi, z2d)
  out = _tc2(sums2, degp, h_lo, h_hi, W2_rel, b2, W2_root)
  return out[:N_NODES]


# speedup vs baseline: 8.3554x; 1.0002x over previous
import functools

import jax
import jax.numpy as jnp
from jax import lax
from jax.experimental import pallas as pl
from jax.experimental.pallas import tpu as pltpu
from jax.experimental.pallas import tpu_sc as plsc

N_NODES = 10000
N_EDGES = 320000
D_IN = 128
D_HID = 256

NC = 2
NS = 16
NW = NC * NS

NPAD = 10240
GB = 256
NBLK = N_EDGES // GB
ROWS_PER_TILE = NPAD // NS

_mesh = plsc.VectorSubcoreMesh(core_axis_name="c", subcore_axis_name="s")


def _zero_acc_slice(z2d_hbm, acc, s):
  row0 = s * ROWS_PER_TILE
  pltpu.sync_copy(z2d_hbm.at[pl.ds(row0, ROWS_PER_TILE), :],
                  acc.at[pl.ds(row0, ROWS_PER_TILE), :])


W = 128


def _agg_loop(eidx_hbm, tbl_hbm, acc, eidx_v, rows_v, gsem, ssem,
              blk0, blk_end, deg_v=None):
  ones16 = jnp.ones((16,), jnp.float32)

  def block(b, carry):
    pltpu.sync_copy(eidx_hbm.at[b], eidx_v)
    g0 = pltpu.async_copy(tbl_hbm.at[eidx_v.at[0]], rows_v.at[0], gsem.at[0])
    g1 = pltpu.async_copy(tbl_hbm.at[eidx_v.at[1]], rows_v.at[1], gsem.at[1])
    g0.wait()
    s0 = pltpu.async_copy(rows_v.at[0], acc.at[eidx_v.at[2]], ssem.at[0],
                          add=True)
    g1.wait()
    s1 = pltpu.async_copy(rows_v.at[1], acc.at[eidx_v.at[3]], ssem.at[1],
                          add=True)
    if deg_v is not None:
      for r in (2, 3):
        for j in range(W // 16):
          dvec = eidx_v[r, pl.ds(j * 16, 16)]
          plsc.addupdate_scatter(deg_v, [dvec], ones16)
    s0.wait()
    s1.wait()
    return carry

  lax.fori_loop(blk0, blk_end, block, 0)


def _sc_agg1_body(eidx_hbm, x_hbm, z2d_hbm, z1d_hbm, sums_hbm, degs_hbm,
                  eidx_v, rows_v, deg_v, acc, gsem, ssem):
  c = lax.axis_index("c")
  s = lax.axis_index("s")

  pltpu.sync_copy(z1d_hbm, deg_v)
  _zero_acc_slice(z2d_hbm, acc, s)
  plsc.subcore_barrier()

  w = c * NS + s
  _agg_loop(eidx_hbm, x_hbm, acc, eidx_v, rows_v, gsem, ssem,
            w * NBLK // NW, (w + 1) * NBLK // NW, deg_v=deg_v)

  plsc.subcore_barrier()
  row0 = s * ROWS_PER_TILE
  pltpu.sync_copy(acc.at[pl.ds(row0, ROWS_PER_TILE), :],
                  sums_hbm.at[c, pl.ds(row0, ROWS_PER_TILE), :])
  pltpu.sync_copy(deg_v, degs_hbm.at[c * NS + s])


_sc_agg1 = functools.partial(
    pl.kernel,
    out_type=(
        jax.ShapeDtypeStruct((NC, NPAD, D_IN), jnp.float32),
        jax.ShapeDtypeStruct((NW, NPAD), jnp.float32),
    ),
    mesh=_mesh,
    compiler_params=pltpu.CompilerParams(needs_layout_passes=False),
    scratch_types=[
        pltpu.VMEM((4, W), jnp.int32),
        pltpu.VMEM((2, W, D_IN), jnp.float32),
        pltpu.VMEM((NPAD,), jnp.float32),
        pltpu.VMEM_SHARED((NPAD, D_IN), jnp.float32),
        pltpu.SemaphoreType.DMA((2,)),
        pltpu.SemaphoreType.DMA((2,)),
    ],
)(_sc_agg1_body)


def _sc_agg2_body(eidx_hbm, hlo_hbm, hhi_hbm, z2d_hbm, sums_hbm,
                  eidx_v, rows_v, acc, gsem, ssem):
  c = lax.axis_index("c")
  s = lax.axis_index("s")

  _zero_acc_slice(z2d_hbm, acc, s)
  plsc.subcore_barrier()

  blk0 = s * NBLK // NS
  blk_end = (s + 1) * NBLK // NS

  @pl.when(c == 0)
  def _():
    _agg_loop(eidx_hbm, hlo_hbm, acc, eidx_v, rows_v, gsem, ssem,
              blk0, blk_end)

  @pl.when(c == 1)
  def _():
    _agg_loop(eidx_hbm, hhi_hbm, acc, eidx_v, rows_v, gsem, ssem,
              blk0, blk_end)

  plsc.subcore_barrier()
  row0 = s * ROWS_PER_TILE
  pltpu.sync_copy(acc.at[pl.ds(row0, ROWS_PER_TILE), :],
                  sums_hbm.at[c, pl.ds(row0, ROWS_PER_TILE), :])


_sc_agg2 = functools.partial(
    pl.kernel,
    out_type=jax.ShapeDtypeStruct((NC, NPAD, D_IN), jnp.float32),
    mesh=_mesh,
    scratch_types=[
        pltpu.VMEM((4, W), jnp.int32),
        pltpu.VMEM((2, W, D_IN), jnp.float32),
        pltpu.VMEM_SHARED((NPAD, D_IN), jnp.float32),
        pltpu.SemaphoreType.DMA((2,)),
        pltpu.SemaphoreType.DMA((2,)),
    ],
)(_sc_agg2_body)


BLK = 1000


def _tcdeg_body(degp_ref, inv_ref):
  deg = jnp.sum(degp_ref[...], axis=0)
  inv_ref[...] = (1.0 / jnp.maximum(deg, 1.0))[:, None]


def _tcdeg(degp):
  return pl.pallas_call(
      _tcdeg_body,
      out_shape=jax.ShapeDtypeStruct((NPAD, 1), jnp.float32),
  )(degp)


def _tc1_body(x_ref, sa_ref, sb_ref, inv_ref, w1rel_ref, b1_ref, w1root_ref,
              lo_ref, hi_ref):
  mean = (sa_ref[...][0] + sb_ref[...][0]) * inv_ref[...]
  h = jnp.dot(mean, w1rel_ref[...], preferred_element_type=jnp.float32)
  h = h + jnp.dot(x_ref[...], w1root_ref[...], preferred_element_type=jnp.float32)
  h = h + b1_ref[...]
  h = jnp.maximum(h, 0.0)
  lo_ref[...] = h[:, :D_IN]
  hi_ref[...] = h[:, D_IN:]


def _tc2_body(mlo_ref, mhi_ref, inv_ref, hlo_ref, hhi_ref,
              w2rel_ref, b2_ref, w2root_ref, out_ref):
  mean = jnp.concatenate([mlo_ref[...][0], mhi_ref[...][0]], axis=1) * inv_ref[...]
  h = jnp.concatenate([hlo_ref[...], hhi_ref[...]], axis=1)
  out = jnp.dot(mean, w2rel_ref[...], preferred_element_type=jnp.float32)
  out = out + jnp.dot(h, w2root_ref[...], preferred_element_type=jnp.float32)
  out_ref[...] = out + b2_ref[...]


def _tc1(x, sums1, inv, W1_rel, b1, W1_root):
  grid = (N_NODES // BLK,)
  return pl.pallas_call(
      _tc1_body,
      grid=grid,
      in_specs=[
          pl.BlockSpec((BLK, D_IN), lambda i: (i, 0)),
          pl.BlockSpec((1, BLK, D_IN), lambda i: (0, i, 0)),
          pl.BlockSpec((1, BLK, D_IN), lambda i: (1, i, 0)),
          pl.BlockSpec((BLK, 1), lambda i: (i, 0)),
          pl.BlockSpec((D_IN, D_HID), lambda i: (0, 0)),
          pl.BlockSpec((1, D_HID), lambda i: (0, 0)),
          pl.BlockSpec((D_IN, D_HID), lambda i: (0, 0)),
      ],
      out_specs=[
          pl.BlockSpec((BLK, D_IN), lambda i: (i, 0)),
          pl.BlockSpec((BLK, D_IN), lambda i: (i, 0)),
      ],
      out_shape=[
          jax.ShapeDtypeStruct((N_NODES, D_IN), jnp.float32),
          jax.ShapeDtypeStruct((N_NODES, D_IN), jnp.float32),
      ],
  )(x, sums1, sums1, inv, W1_rel, b1.reshape(1, D_HID), W1_root)


def _tc2(sums2, inv, h_lo, h_hi, W2_rel, b2, W2_root):
  grid = (N_NODES // BLK,)
  return pl.pallas_call(
      _tc2_body,
      grid=grid,
      in_specs=[
          pl.BlockSpec((1, BLK, D_IN), lambda i: (0, i, 0)),
          pl.BlockSpec((1, BLK, D_IN), lambda i: (1, i, 0)),
          pl.BlockSpec((BLK, 1), lambda i: (i, 0)),
          pl.BlockSpec((BLK, D_IN), lambda i: (i, 0)),
          pl.BlockSpec((BLK, D_IN), lambda i: (i, 0)),
          pl.BlockSpec((D_HID, D_HID), lambda i: (0, 0)),
          pl.BlockSpec((1, D_HID), lambda i: (0, 0)),
          pl.BlockSpec((D_HID, D_HID), lambda i: (0, 0)),
      ],
      out_specs=pl.BlockSpec((BLK, D_HID), lambda i: (i, 0)),
      out_shape=jax.ShapeDtypeStruct((N_NODES, D_HID), jnp.float32),
  )(sums2, sums2, inv, h_lo, h_hi, W2_rel, b2.reshape(1, D_HID), W2_root)


def kernel(x, edge_index, W1_rel, b1, W1_root, W2_rel, b2, W2_root):
  src = edge_index[0].astype(jnp.int32)
  dst = edge_index[1].astype(jnp.int32)
  srcb = src.reshape(NBLK, 2, W)
  dstb = dst.reshape(NBLK, 2, W)
  eidx = jnp.concatenate([srcb, dstb], axis=1)

  z2d = jnp.zeros((NPAD, D_IN), jnp.float32)
  z1d = jnp.zeros((NPAD,), jnp.float32)
  sums1, degp = _sc_agg1(eidx, x, z2d, z1d)
  inv = _tcdeg(degp)
  h_lo, h_hi = _tc1(x, sums1, inv, W1_rel, b1, W1_root)
  sums2 = _sc_agg2(eidx, h_lo, h_hi, z2d)
  return _tc2(sums2, inv, h_lo, h_hi, W2_rel, b2, W2_root)
